# Initial kernel scaffold; baseline (speedup 1.0000x reference)
import jax
import jax.numpy as jnp
from jax.experimental import pallas as pl


def kernel(x0, x1, x2, x3, x4, x5, x6, x7, edge_index, params):
    raise NotImplementedError("write your pallas kernel here")


# speedup vs baseline: 3.4780x; 3.4780x over previous
import functools

import jax
import jax.numpy as jnp
import numpy as np
from jax import lax
from jax.experimental import pallas as pl
from jax.experimental.pallas import tpu as pltpu
from jax.experimental.pallas import tpu_sc as plsc

N = 10000
E = 320000
H = 8
DH = 16
HID = 128
OUT = 64
V = 8

NC = 2
NS = 16
NW = NC * NS
EW = E // NW
C = 80
PW = HID + 16
NP = 10240
RPS = NP // NS

BN = 1000
BE = 4000
BA = 512

_sc_mesh = plsc.VectorSubcoreMesh(core_axis_name="c", subcore_axis_name="s")


@functools.partial(
    pl.kernel,
    out_type=(jax.ShapeDtypeStruct((E, HID), jnp.float32),
              jax.ShapeDtypeStruct((E, 2 * HID), jnp.float32)),
    mesh=_sc_mesh,
    scratch_types=[pltpu.VMEM((C,), jnp.int32),
                   pltpu.VMEM((C, HID), jnp.float32),
                   pltpu.VMEM((C,), jnp.int32),
                   pltpu.VMEM((C, 2 * HID), jnp.float32),
                   pltpu.SemaphoreType.DMA,
                   pltpu.SemaphoreType.DMA],
)
def _sc_gather(qtab, kvtab, dsti, srci, qq, kvg, idxd, qrows, idxs, kvrows,
               sem1, sem2):
    wid = lax.axis_index("s") * NC + lax.axis_index("c")

    def step(i, carry):
        base = wid * EW + i * C
        pltpu.sync_copy(dsti.at[pl.ds(base, C)], idxd)
        pltpu.sync_copy(srci.at[pl.ds(base, C)], idxs)
        cp1 = pltpu.async_copy(qtab.at[idxd], qrows, sem1)
        cp2 = pltpu.async_copy(kvtab.at[idxs], kvrows, sem2)
        cp1.wait()
        cp2.wait()
        pltpu.sync_copy(qrows, qq.at[pl.ds(base, C)])
        pltpu.sync_copy(kvrows, kvg.at[pl.ds(base, C)])
        return carry

    lax.fori_loop(0, EW // C, step, 0)


@functools.partial(
    pl.kernel,
    out_type=jax.ShapeDtypeStruct((NC, NP, PW), jnp.float32),
    mesh=_sc_mesh,
    scratch_types=[pltpu.VMEM((C,), jnp.int32),
                   pltpu.VMEM((C, PW), jnp.float32),
                   pltpu.VMEM_SHARED((NP, PW), jnp.float32)],
    compiler_params=pltpu.CompilerParams(use_tc_tiling_on_sc=False),
)
def _sc_scatter_add(payload, dsti, zrows, out, idx_v, rows, acc):
    c = lax.axis_index("c")
    s = lax.axis_index("s")
    wid = s * NC + c
    pltpu.sync_copy(zrows.at[pl.ds(s * RPS, RPS)], acc.at[pl.ds(s * RPS, RPS)])
    plsc.subcore_barrier()

    def step(i, carry):
        base = wid * EW + i * C
        pltpu.sync_copy(dsti.at[pl.ds(base, C)], idx_v)
        pltpu.sync_copy(payload.at[pl.ds(base, C)], rows)
        pltpu.sync_copy(rows, acc.at[idx_v], add=True)
        return carry

    lax.fori_loop(0, EW // C, step, 0)
    plsc.subcore_barrier()
    pltpu.sync_copy(acc.at[pl.ds(s * RPS, RPS)],
                    out.at[c, pl.ds(s * RPS, RPS)])


def _mm_relu(x, W, b):
    def body(x_r, w_r, b_r, o_r):
        o_r[...] = jax.nn.relu(
            jnp.dot(x_r[...], w_r[...], preferred_element_type=jnp.float32)
            + b_r[...])
    return pl.pallas_call(
        body,
        grid=(N // BN,),
        in_specs=[pl.BlockSpec((BN, HID), lambda i: (i, 0)),
                  pl.BlockSpec((HID, HID), lambda i: (0, 0)),
                  pl.BlockSpec((1, HID), lambda i: (0, 0))],
        out_specs=pl.BlockSpec((BN, HID), lambda i: (i, 0)),
        out_shape=jax.ShapeDtypeStruct((N, HID), jnp.float32),
    )(x, W, b)


def _qkv(h, Wc, bc):
    def body(h_r, w_r, b_r, q_r, kv_r):
        y = (jnp.dot(h_r[...], w_r[...], preferred_element_type=jnp.float32)
             + b_r[...])
        q_r[...] = y[:, :HID]
        kv_r[...] = y[:, HID:]
    return pl.pallas_call(
        body,
        grid=(N // BN,),
        in_specs=[pl.BlockSpec((BN, HID), lambda i: (i, 0)),
                  pl.BlockSpec((HID, 3 * HID), lambda i: (0, 0)),
                  pl.BlockSpec((1, 3 * HID), lambda i: (0, 0))],
        out_specs=[pl.BlockSpec((BN, HID), lambda i: (i, 0)),
                   pl.BlockSpec((BN, 2 * HID), lambda i: (i, 0))],
        out_shape=[jax.ShapeDtypeStruct((N, HID), jnp.float32),
                   jax.ShapeDtypeStruct((N, 2 * HID), jnp.float32)],
    )(h, Wc, bc)


def _scores(qq, kvg):
    def body(q_r, k_r, s_r, m_r):
        i = pl.program_id(0)
        s = (q_r[...].reshape(BE, H, DH) * k_r[...].reshape(BE, H, DH)).sum(-1)
        s_r[...] = s
        m = jnp.max(s, axis=0, keepdims=True)

        @pl.when(i == 0)
        def _():
            m_r[...] = m

        @pl.when(i > 0)
        def _():
            m_r[...] = jnp.maximum(m_r[...], m)

    return pl.pallas_call(
        body,
        grid=(E // BE,),
        in_specs=[pl.BlockSpec((BE, HID), lambda i: (i, 0)),
                  pl.BlockSpec((BE, HID), lambda i: (i, 0))],
        out_specs=[pl.BlockSpec((BE, H), lambda i: (i, 0)),
                   pl.BlockSpec((1, H), lambda i: (0, 0))],
        out_shape=[jax.ShapeDtypeStruct((E, H), jnp.float32),
                   jax.ShapeDtypeStruct((1, H), jnp.float32)],
    )(qq, kvg)


def _payload(scores, gmax, kvg):
    def body(s_r, m_r, v_r, p_r):
        e = jnp.exp(s_r[...] - m_r[...])
        msg = (v_r[...].reshape(BE, H, DH) * e[:, :, None]).reshape(BE, HID)
        p_r[...] = jnp.concatenate(
            [msg, e, jnp.zeros((BE, PW - HID - H), jnp.float32)], axis=1)
    return pl.pallas_call(
        body,
        grid=(E // BE,),
        in_specs=[pl.BlockSpec((BE, H), lambda i: (i, 0)),
                  pl.BlockSpec((1, H), lambda i: (0, 0)),
                  pl.BlockSpec((BE, HID), lambda i: (i, 1))],
        out_specs=pl.BlockSpec((BE, PW), lambda i: (i, 0)),
        out_shape=jax.ShapeDtypeStruct((E, PW), jnp.float32),
    )(scores, gmax, kvg)


def _combine(partials, h, aW, ab, beta_row):
    def body(p0_r, p1_r, h_r, w_r, b_r, bt_r, o_r):
        acc = p0_r[0] + p1_r[0]
        den = acc[:, HID:HID + H] + 1e-16
        msg = acc[:, :HID].reshape(BN, H, DH) / den[:, :, None]
        g = jax.nn.gelu(msg.reshape(BN, HID))
        o = (jnp.dot(g, w_r[...], preferred_element_type=jnp.float32)
             + b_r[...])
        bt = bt_r[...]
        o_r[...] = bt * o + (1.0 - bt) * h_r[...]

    return pl.pallas_call(
        body,
        grid=(N // BN,),
        in_specs=[pl.BlockSpec((1, BN, PW), lambda i: (0, i, 0)),
                  pl.BlockSpec((1, BN, PW), lambda i: (1, i, 0)),
                  pl.BlockSpec((BN, HID), lambda i: (i, 0)),
                  pl.BlockSpec((HID, HID), lambda i: (0, 0)),
                  pl.BlockSpec((1, HID), lambda i: (0, 0)),
                  pl.BlockSpec((1, HID), lambda i: (0, 0))],
        out_specs=pl.BlockSpec((BN, HID), lambda i: (i, 0)),
        out_shape=jax.ShapeDtypeStruct((N, HID), jnp.float32),
    )(partials, partials, h, aW, ab, beta_row)


def _mix_out(hsS, wraw, oW, ob):
    def body(hs_r, w_r, ow_r, ob_r, z_r):
        w = jax.nn.softmax(w_r[...], axis=-1).reshape(V, 1, 1)
        hbar = (hs_r[...] * w).sum(0)
        z_r[...] = (jnp.dot(hbar, ow_r[...],
                            preferred_element_type=jnp.float32) + ob_r[...])
    return pl.pallas_call(
        body,
        grid=(N // BN,),
        in_specs=[pl.BlockSpec((V, BN, HID), lambda i: (0, i, 0)),
                  pl.BlockSpec((1, V), lambda i: (0, 0)),
                  pl.BlockSpec((HID, OUT), lambda i: (0, 0)),
                  pl.BlockSpec((1, OUT), lambda i: (0, 0))],
        out_specs=pl.BlockSpec((BN, OUT), lambda i: (i, 0)),
        out_shape=jax.ShapeDtypeStruct((N, OUT), jnp.float32),
    )(hsS, wraw, oW, ob)


def _head(Z, lW, lb, tW, tb, wraw, wtraw):
    def body(z_r, lw_r, lb_r, tw_r, tb_r, wr_r, wtr_r, x_r, t_r, w_r, wt_r):
        X = (jnp.dot(z_r[...], lw_r[...], preferred_element_type=jnp.float32)
             + lb_r[...])
        x_r[...] = X
        wt = jax.nn.softmax(wtr_r[...], axis=-1)
        t = (jnp.dot(X, tw_r[...], preferred_element_type=jnp.float32)
             + tb_r[...]) * wt
        t_r[...] = jax.nn.softmax(t, axis=0)
        w_r[...] = jax.nn.softmax(wr_r[...], axis=-1)
        wt_r[...] = wt
    return pl.pallas_call(
        body,
        out_shape=[jax.ShapeDtypeStruct((N, HID), jnp.float32),
                   jax.ShapeDtypeStruct((N, 2), jnp.float32),
                   jax.ShapeDtypeStruct((1, V), jnp.float32),
                   jax.ShapeDtypeStruct((1, 2), jnp.float32)],
    )(Z, lW, lb, tW, tb, wraw, wtraw)


def _adjacency(Z):
    def body(zi_r, zj_r, a_r):
        a_r[...] = jax.nn.sigmoid(
            lax.dot_general(zi_r[...], zj_r[...], (((1,), (1,)), ((), ())),
                            preferred_element_type=jnp.float32))
    G = pl.cdiv(N, BA)
    return pl.pallas_call(
        body,
        grid=(G, G),
        in_specs=[pl.BlockSpec((BA, OUT), lambda i, j: (i, 0)),
                  pl.BlockSpec((BA, OUT), lambda i, j: (j, 0))],
        out_specs=pl.BlockSpec((BA, BA), lambda i, j: (i, j)),
        out_shape=jax.ShapeDtypeStruct((N, N), jnp.float32),
    )(Z, Z)


def _fold_conv_weights(cp):
    arel = cp['a_rel'] * (cp['p_rel'] / np.sqrt(DH))[:, None, None]
    Keff = jnp.einsum('ihd,hde->ihe', cp['k_W'].reshape(HID, H, DH),
                      arel).reshape(HID, HID)
    kbe = jnp.einsum('hd,hde->he', cp['k_b'].reshape(H, DH),
                     arel).reshape(HID)
    Veff = jnp.einsum('ihd,hde->ihe', cp['v_W'].reshape(HID, H, DH),
                      cp['m_rel']).reshape(HID, HID)
    vbe = jnp.einsum('hd,hde->he', cp['v_b'].reshape(H, DH),
                     cp['m_rel']).reshape(HID)
    Wc = jnp.concatenate([cp['q_W'], Keff, Veff], axis=1)
    bc = jnp.concatenate([cp['q_b'], kbe, vbe]).reshape(1, 3 * HID)
    beta_row = jnp.full((1, HID), jax.nn.sigmoid(cp['skip']), jnp.float32)
    return Wc, bc, beta_row


def kernel(x0, x1, x2, x3, x4, x5, x6, x7, edge_index, params):
    xs = (x0, x1, x2, x3, x4, x5, x6, x7)
    src = edge_index[0]
    dst = edge_index[1]
    zrows = jnp.zeros((NP, PW), jnp.float32)

    folded = [_fold_conv_weights(cp) for cp in params['convs']]
    a_proj = [(cp['a_W'], cp['a_b'].reshape(1, HID))
              for cp in params['convs']]

    hs = []
    for v in range(V):
        h = _mm_relu(xs[v], params['lin0_W'][v],
                     params['lin0_b'][v].reshape(1, HID))
        for l, (Wc, bc, beta_row) in enumerate(folded):
            q, kv = _qkv(h, Wc, bc)
            qq, kvg = _sc_gather(q, kv, dst, src)
            scores, gmax = _scores(qq, kvg)
            payload = _payload(scores, gmax, kvg)
            partials = _sc_scatter_add(payload, dst, zrows)
            aW, ab = a_proj[l]
            h = _combine(partials, h, aW, ab, beta_row)
        hs.append(h)

    hsS = jnp.stack(hs)
    Z = _mix_out(hsS, params['weight'].reshape(1, V),
                 params['out_W'], params['out_b'].reshape(1, OUT))
    X, T, w2, wt2 = _head(Z, params['lin_W'], params['lin_b'].reshape(1, HID),
                          params['T_W'], params['T_b'].reshape(1, 2),
                          params['weight'].reshape(1, V),
                          params['weight_node_type'].reshape(1, 2))
    A = _adjacency(Z)
    return (A, X, T, w2.reshape(V), wt2.reshape(2))


# speedup vs baseline: 3.4817x; 1.0011x over previous
import functools

import jax
import jax.numpy as jnp
import numpy as np
from jax import lax
from jax.experimental import pallas as pl
from jax.experimental.pallas import tpu as pltpu
from jax.experimental.pallas import tpu_sc as plsc

N = 10000
E = 320000
H = 8
DH = 16
HID = 128
OUT = 64
V = 8

NC = 2
NS = 16
NW = NC * NS
EW = E // NW
C = 80
PW = HID + 16
NP = 10240
RPS = NP // NS

BN = 1000
BE = 4000
BA = 512

_sc_mesh = plsc.VectorSubcoreMesh(core_axis_name="c", subcore_axis_name="s")


@functools.partial(
    pl.kernel,
    out_type=(jax.ShapeDtypeStruct((E, HID), jnp.float32),
              jax.ShapeDtypeStruct((E, 2 * HID), jnp.float32)),
    mesh=_sc_mesh,
    scratch_types=[pltpu.VMEM((2, C), jnp.int32),
                   pltpu.VMEM((2, C, HID), jnp.float32),
                   pltpu.VMEM((2, C), jnp.int32),
                   pltpu.VMEM((2, C, 2 * HID), jnp.float32),
                   pltpu.SemaphoreType.DMA,
                   pltpu.SemaphoreType.DMA,
                   pltpu.SemaphoreType.DMA,
                   pltpu.SemaphoreType.DMA],
)
def _sc_gather(qtab, kvtab, dsti, srci, qq, kvg, idxd, qrows, idxs, kvrows,
               semq0, semq1, semk0, semk1):
    wid = lax.axis_index("s") * NC + lax.axis_index("c")
    nchunks = EW // C
    semq = (semq0, semq1)
    semk = (semk0, semk1)

    def fire(i, b):
        base = wid * EW + i * C
        pltpu.sync_copy(dsti.at[pl.ds(base, C)], idxd.at[b])
        pltpu.sync_copy(srci.at[pl.ds(base, C)], idxs.at[b])
        pltpu.async_copy(qtab.at[idxd.at[b]], qrows.at[b], semq[b])
        pltpu.async_copy(kvtab.at[idxs.at[b]], kvrows.at[b], semk[b])

    def drain(i, b):
        base = wid * EW + i * C
        pltpu.make_async_copy(qtab.at[idxd.at[b]], qrows.at[b], semq[b]).wait()
        pltpu.make_async_copy(kvtab.at[idxs.at[b]], kvrows.at[b],
                              semk[b]).wait()
        pltpu.sync_copy(qrows.at[b], qq.at[pl.ds(base, C)])
        pltpu.sync_copy(kvrows.at[b], kvg.at[pl.ds(base, C)])

    fire(0, 0)

    def step(p, carry):
        i = 2 * p
        fire(i + 1, 1)
        drain(i, 0)

        @pl.when(i + 2 < nchunks)
        def _():
            fire(i + 2, 0)

        drain(i + 1, 1)
        return carry

    lax.fori_loop(0, nchunks // 2, step, 0)
    drain(nchunks - 1, 0)


@functools.partial(
    pl.kernel,
    out_type=jax.ShapeDtypeStruct((NC, NP, PW), jnp.float32),
    mesh=_sc_mesh,
    scratch_types=[pltpu.VMEM((C,), jnp.int32),
                   pltpu.VMEM((C, PW), jnp.float32),
                   pltpu.VMEM_SHARED((NP, PW), jnp.float32)],
    compiler_params=pltpu.CompilerParams(use_tc_tiling_on_sc=False),
)
def _sc_scatter_add(payload, dsti, zrows, out, idx_v, rows, acc):
    c = lax.axis_index("c")
    s = lax.axis_index("s")
    wid = s * NC + c
    pltpu.sync_copy(zrows.at[pl.ds(s * RPS, RPS)], acc.at[pl.ds(s * RPS, RPS)])
    plsc.subcore_barrier()

    def step(i, carry):
        base = wid * EW + i * C
        pltpu.sync_copy(dsti.at[pl.ds(base, C)], idx_v)
        pltpu.sync_copy(payload.at[pl.ds(base, C)], rows)
        pltpu.sync_copy(rows, acc.at[idx_v], add=True)
        return carry

    lax.fori_loop(0, EW // C, step, 0)
    plsc.subcore_barrier()
    pltpu.sync_copy(acc.at[pl.ds(s * RPS, RPS)],
                    out.at[c, pl.ds(s * RPS, RPS)])


def _mm_relu(x, W, b):
    def body(x_r, w_r, b_r, o_r):
        o_r[...] = jax.nn.relu(
            jnp.dot(x_r[...], w_r[...], preferred_element_type=jnp.float32)
            + b_r[...])
    return pl.pallas_call(
        body,
        grid=(N // BN,),
        in_specs=[pl.BlockSpec((BN, HID), lambda i: (i, 0)),
                  pl.BlockSpec((HID, HID), lambda i: (0, 0)),
                  pl.BlockSpec((1, HID), lambda i: (0, 0))],
        out_specs=pl.BlockSpec((BN, HID), lambda i: (i, 0)),
        out_shape=jax.ShapeDtypeStruct((N, HID), jnp.float32),
    )(x, W, b)


def _qkv(h, Wc, bc):
    def body(h_r, w_r, b_r, q_r, kv_r):
        y = (jnp.dot(h_r[...], w_r[...], preferred_element_type=jnp.float32)
             + b_r[...])
        q_r[...] = y[:, :HID]
        kv_r[...] = y[:, HID:]
    return pl.pallas_call(
        body,
        grid=(N // BN,),
        in_specs=[pl.BlockSpec((BN, HID), lambda i: (i, 0)),
                  pl.BlockSpec((HID, 3 * HID), lambda i: (0, 0)),
                  pl.BlockSpec((1, 3 * HID), lambda i: (0, 0))],
        out_specs=[pl.BlockSpec((BN, HID), lambda i: (i, 0)),
                   pl.BlockSpec((BN, 2 * HID), lambda i: (i, 0))],
        out_shape=[jax.ShapeDtypeStruct((N, HID), jnp.float32),
                   jax.ShapeDtypeStruct((N, 2 * HID), jnp.float32)],
    )(h, Wc, bc)


def _scores(qq, kvg):
    def body(q_r, k_r, s_r, m_r):
        i = pl.program_id(0)
        s = (q_r[...].reshape(BE, H, DH) * k_r[...].reshape(BE, H, DH)).sum(-1)
        s_r[...] = s
        m = jnp.max(s, axis=0, keepdims=True)

        @pl.when(i == 0)
        def _():
            m_r[...] = m

        @pl.when(i > 0)
        def _():
            m_r[...] = jnp.maximum(m_r[...], m)

    return pl.pallas_call(
        body,
        grid=(E // BE,),
        in_specs=[pl.BlockSpec((BE, HID), lambda i: (i, 0)),
                  pl.BlockSpec((BE, HID), lambda i: (i, 0))],
        out_specs=[pl.BlockSpec((BE, H), lambda i: (i, 0)),
                   pl.BlockSpec((1, H), lambda i: (0, 0))],
        out_shape=[jax.ShapeDtypeStruct((E, H), jnp.float32),
                   jax.ShapeDtypeStruct((1, H), jnp.float32)],
    )(qq, kvg)


def _payload(scores, gmax, kvg):
    def body(s_r, m_r, v_r, p_r):
        e = jnp.exp(s_r[...] - m_r[...])
        msg = (v_r[...].reshape(BE, H, DH) * e[:, :, None]).reshape(BE, HID)
        p_r[...] = jnp.concatenate(
            [msg, e, jnp.zeros((BE, PW - HID - H), jnp.float32)], axis=1)
    return pl.pallas_call(
        body,
        grid=(E // BE,),
        in_specs=[pl.BlockSpec((BE, H), lambda i: (i, 0)),
                  pl.BlockSpec((1, H), lambda i: (0, 0)),
                  pl.BlockSpec((BE, HID), lambda i: (i, 1))],
        out_specs=pl.BlockSpec((BE, PW), lambda i: (i, 0)),
        out_shape=jax.ShapeDtypeStruct((E, PW), jnp.float32),
    )(scores, gmax, kvg)


def _combine(partials, h, aW, ab, beta_row):
    def body(p0_r, p1_r, h_r, w_r, b_r, bt_r, o_r):
        acc = p0_r[0] + p1_r[0]
        den = acc[:, HID:HID + H] + 1e-16
        msg = acc[:, :HID].reshape(BN, H, DH) / den[:, :, None]
        g = jax.nn.gelu(msg.reshape(BN, HID))
        o = (jnp.dot(g, w_r[...], preferred_element_type=jnp.float32)
             + b_r[...])
        bt = bt_r[...]
        o_r[...] = bt * o + (1.0 - bt) * h_r[...]

    return pl.pallas_call(
        body,
        grid=(N // BN,),
        in_specs=[pl.BlockSpec((1, BN, PW), lambda i: (0, i, 0)),
                  pl.BlockSpec((1, BN, PW), lambda i: (1, i, 0)),
                  pl.BlockSpec((BN, HID), lambda i: (i, 0)),
                  pl.BlockSpec((HID, HID), lambda i: (0, 0)),
                  pl.BlockSpec((1, HID), lambda i: (0, 0)),
                  pl.BlockSpec((1, HID), lambda i: (0, 0))],
        out_specs=pl.BlockSpec((BN, HID), lambda i: (i, 0)),
        out_shape=jax.ShapeDtypeStruct((N, HID), jnp.float32),
    )(partials, partials, h, aW, ab, beta_row)


def _mix_out(hsS, wraw, oW, ob):
    def body(hs_r, w_r, ow_r, ob_r, z_r):
        w = jax.nn.softmax(w_r[...], axis=-1).reshape(V, 1, 1)
        hbar = (hs_r[...] * w).sum(0)
        z_r[...] = (jnp.dot(hbar, ow_r[...],
                            preferred_element_type=jnp.float32) + ob_r[...])
    return pl.pallas_call(
        body,
        grid=(N // BN,),
        in_specs=[pl.BlockSpec((V, BN, HID), lambda i: (0, i, 0)),
                  pl.BlockSpec((1, V), lambda i: (0, 0)),
                  pl.BlockSpec((HID, OUT), lambda i: (0, 0)),
                  pl.BlockSpec((1, OUT), lambda i: (0, 0))],
        out_specs=pl.BlockSpec((BN, OUT), lambda i: (i, 0)),
        out_shape=jax.ShapeDtypeStruct((N, OUT), jnp.float32),
    )(hsS, wraw, oW, ob)


def _head(Z, lW, lb, tW, tb, wraw, wtraw):
    def body(z_r, lw_r, lb_r, tw_r, tb_r, wr_r, wtr_r, x_r, t_r, w_r, wt_r):
        X = (jnp.dot(z_r[...], lw_r[...], preferred_element_type=jnp.float32)
             + lb_r[...])
        x_r[...] = X
        wt = jax.nn.softmax(wtr_r[...], axis=-1)
        t = (jnp.dot(X, tw_r[...], preferred_element_type=jnp.float32)
             + tb_r[...]) * wt
        t_r[...] = jax.nn.softmax(t, axis=0)
        w_r[...] = jax.nn.softmax(wr_r[...], axis=-1)
        wt_r[...] = wt
    return pl.pallas_call(
        body,
        out_shape=[jax.ShapeDtypeStruct((N, HID), jnp.float32),
                   jax.ShapeDtypeStruct((N, 2), jnp.float32),
                   jax.ShapeDtypeStruct((1, V), jnp.float32),
                   jax.ShapeDtypeStruct((1, 2), jnp.float32)],
    )(Z, lW, lb, tW, tb, wraw, wtraw)


def _adjacency(Z):
    def body(zi_r, zj_r, a_r):
        a_r[...] = jax.nn.sigmoid(
            lax.dot_general(zi_r[...], zj_r[...], (((1,), (1,)), ((), ())),
                            preferred_element_type=jnp.float32))
    G = pl.cdiv(N, BA)
    return pl.pallas_call(
        body,
        grid=(G, G),
        in_specs=[pl.BlockSpec((BA, OUT), lambda i, j: (i, 0)),
                  pl.BlockSpec((BA, OUT), lambda i, j: (j, 0))],
        out_specs=pl.BlockSpec((BA, BA), lambda i, j: (i, j)),
        out_shape=jax.ShapeDtypeStruct((N, N), jnp.float32),
    )(Z, Z)


def _fold_conv_weights(cp):
    arel = cp['a_rel'] * (cp['p_rel'] / np.sqrt(DH))[:, None, None]
    Keff = jnp.einsum('ihd,hde->ihe', cp['k_W'].reshape(HID, H, DH),
                      arel).reshape(HID, HID)
    kbe = jnp.einsum('hd,hde->he', cp['k_b'].reshape(H, DH),
                     arel).reshape(HID)
    Veff = jnp.einsum('ihd,hde->ihe', cp['v_W'].reshape(HID, H, DH),
                      cp['m_rel']).reshape(HID, HID)
    vbe = jnp.einsum('hd,hde->he', cp['v_b'].reshape(H, DH),
                     cp['m_rel']).reshape(HID)
    Wc = jnp.concatenate([cp['q_W'], Keff, Veff], axis=1)
    bc = jnp.concatenate([cp['q_b'], kbe, vbe]).reshape(1, 3 * HID)
    beta_row = jnp.full((1, HID), jax.nn.sigmoid(cp['skip']), jnp.float32)
    return Wc, bc, beta_row


def kernel(x0, x1, x2, x3, x4, x5, x6, x7, edge_index, params):
    xs = (x0, x1, x2, x3, x4, x5, x6, x7)
    src = edge_index[0]
    dst = edge_index[1]
    zrows = jnp.zeros((NP, PW), jnp.float32)

    folded = [_fold_conv_weights(cp) for cp in params['convs']]
    a_proj = [(cp['a_W'], cp['a_b'].reshape(1, HID))
              for cp in params['convs']]

    hs = []
    for v in range(V):
        h = _mm_relu(xs[v], params['lin0_W'][v],
                     params['lin0_b'][v].reshape(1, HID))
        for l, (Wc, bc, beta_row) in enumerate(folded):
            q, kv = _qkv(h, Wc, bc)
            qq, kvg = _sc_gather(q, kv, dst, src)
            scores, gmax = _scores(qq, kvg)
            payload = _payload(scores, gmax, kvg)
            partials = _sc_scatter_add(payload, dst, zrows)
            aW, ab = a_proj[l]
            h = _combine(partials, h, aW, ab, beta_row)
        hs.append(h)

    hsS = jnp.stack(hs)
    Z = _mix_out(hsS, params['weight'].reshape(1, V),
                 params['out_W'], params['out_b'].reshape(1, OUT))
    X, T, w2, wt2 = _head(Z, params['lin_W'], params['lin_b'].reshape(1, HID),
                          params['T_W'], params['T_b'].reshape(1, 2),
                          params['weight'].reshape(1, V),
                          params['weight_node_type'].reshape(1, 2))
    A = _adjacency(Z)
    return (A, X, T, w2.reshape(V), wt2.reshape(2))
